# initial kernel scaffold (unmeasured)
import jax
import jax.numpy as jnp
from jax import lax
from jax.experimental import pallas as pl
from jax.experimental.pallas import tpu as pltpu

N_DEV = 4


def _gelu(z):
    return 0.5 * z * (1.0 + jnp.tanh(0.7978845608 * (z + 0.044715 * z * z * z)))


def kernel(A, B):
    m, k = A.shape
    _, n = B.shape

    def body(a_ref, b_ref, out_ref, comm_ref, send_sems, recv_sems):
        my = lax.axis_index("i")
        left = (my + N_DEV - 1) % N_DEV
        right = (my + 1) % N_DEV

        barrier_sem = pltpu.get_barrier_semaphore()
        for nbr in (left, right):
            pl.semaphore_signal(
                barrier_sem, inc=1,
                device_id=(nbr,), device_id_type=pl.DeviceIdType.MESH,
            )
        pl.semaphore_wait(barrier_sem, 2)

        partial = jnp.dot(
            a_ref[...], b_ref[...], preferred_element_type=jnp.float32
        )
        comm_ref[0] = partial
        out_ref[...] = partial

        for h in range(N_DEV - 1):
            ss = h % 2
            rs = (h + 1) % 2
            rdma = pltpu.make_async_remote_copy(
                src_ref=comm_ref.at[ss],
                dst_ref=comm_ref.at[rs],
                send_sem=send_sems.at[ss],
                recv_sem=recv_sems.at[rs],
                device_id=(right,),
                device_id_type=pl.DeviceIdType.MESH,
            )
            rdma.start()
            rdma.wait()
            out_ref[...] += comm_ref[rs]

        out_ref[...] = _gelu(out_ref[...])

    return pl.pallas_call(
        body,
        out_shape=jax.ShapeDtypeStruct((m, n), jnp.float32),
        in_specs=[
            pl.BlockSpec(memory_space=pltpu.VMEM),
            pl.BlockSpec(memory_space=pltpu.VMEM),
        ],
        out_specs=pl.BlockSpec(memory_space=pltpu.VMEM),
        scratch_shapes=[
            pltpu.VMEM((2, m, n), jnp.float32),
            pltpu.SemaphoreType.DMA((2,)),
            pltpu.SemaphoreType.DMA((2,)),
        ],
        compiler_params=pltpu.CompilerParams(collective_id=0),
    )(A, B)


# baseline (device time: 312686 ns/iter reference)
import jax
import jax.numpy as jnp
from jax import lax
from jax.experimental import pallas as pl
from jax.experimental.pallas import tpu as pltpu

N_DEV = 4


def _gelu(z):
    return 0.5 * z * (1.0 + jnp.tanh(0.7978845608 * (z + 0.044715 * z * z * z)))


def kernel(A, B):
    m, k = A.shape
    _, n = B.shape
    mc = m // N_DEV

    def body(a_ref, b_ref, out_ref, rs_comm, ag_comm,
             rs_send, rs_recv, ag_send, ag_recv):
        my = lax.axis_index("i")
        left = (my + N_DEV - 1) % N_DEV
        right = (my + 1) % N_DEV

        barrier_sem = pltpu.get_barrier_semaphore()
        for nbr in (left, right):
            pl.semaphore_signal(
                barrier_sem, inc=1,
                device_id=(nbr,), device_id_type=pl.DeviceIdType.MESH,
            )
        pl.semaphore_wait(barrier_sem, 2)

        def partial(c):
            return jnp.dot(
                a_ref[pl.ds(c * mc, mc), :], b_ref[...],
                preferred_element_type=jnp.float32,
            )

        rs_comm[0] = partial(my)
        for s in range(N_DEV - 1):
            ss = s % 2
            rr = (s + 1) % 2
            rdma = pltpu.make_async_remote_copy(
                src_ref=rs_comm.at[ss],
                dst_ref=rs_comm.at[rr],
                send_sem=rs_send.at[ss],
                recv_sem=rs_recv.at[rr],
                device_id=(right,),
                device_id_type=pl.DeviceIdType.MESH,
            )
            rdma.start()
            p_next = partial((my - s - 1) % N_DEV)
            rdma.wait()
            rs_comm[rr] += p_next

        mine = (my + 1) % N_DEV
        g = _gelu(rs_comm[(N_DEV - 1) % 2])
        out_ref[pl.ds(mine * mc, mc), :] = g
        ag_comm[0] = g

        for t in range(N_DEV - 1):
            ss = t % 2
            rr = (t + 1) % 2
            rdma = pltpu.make_async_remote_copy(
                src_ref=ag_comm.at[ss],
                dst_ref=ag_comm.at[rr],
                send_sem=ag_send.at[ss],
                recv_sem=ag_recv.at[rr],
                device_id=(right,),
                device_id_type=pl.DeviceIdType.MESH,
            )
            rdma.start()
            rdma.wait()
            origin = (my - t) % N_DEV
            out_ref[pl.ds(origin * mc, mc), :] = ag_comm[rr]

    return pl.pallas_call(
        body,
        out_shape=jax.ShapeDtypeStruct((m, n), jnp.float32),
        in_specs=[
            pl.BlockSpec(memory_space=pltpu.VMEM),
            pl.BlockSpec(memory_space=pltpu.VMEM),
        ],
        out_specs=pl.BlockSpec(memory_space=pltpu.VMEM),
        scratch_shapes=[
            pltpu.VMEM((2, mc, n), jnp.float32),
            pltpu.VMEM((2, mc, n), jnp.float32),
            pltpu.SemaphoreType.DMA((2,)),
            pltpu.SemaphoreType.DMA((2,)),
            pltpu.SemaphoreType.DMA((2,)),
            pltpu.SemaphoreType.DMA((2,)),
        ],
        compiler_params=pltpu.CompilerParams(
            collective_id=0,
            vmem_limit_bytes=100 * 1024 * 1024,
        ),
    )(A, B)


# device time: 177891 ns/iter; 1.7577x vs baseline; 1.7577x over previous
import jax
import jax.numpy as jnp
from jax import lax
from jax.experimental import pallas as pl
from jax.experimental.pallas import tpu as pltpu

N_DEV = 4


def _gelu(z):
    return 0.5 * z * (1.0 + jnp.tanh(0.7978845608 * (z + 0.044715 * z * z * z)))


def kernel(A, B):
    m, k = A.shape
    _, n = B.shape
    mc = m // N_DEV
    nh = n // 2

    def body(a_ref, b_ref, out_ref, cw_comm, ccw_comm, agcw_comm, agccw_comm,
             cw_send, cw_recv, ccw_send, ccw_recv,
             agcw_send, agcw_recv, agccw_send, agccw_recv):
        my = lax.axis_index("i")
        left = (my + N_DEV - 1) % N_DEV
        right = (my + 1) % N_DEV

        barrier_sem = pltpu.get_barrier_semaphore()
        for nbr in (left, right):
            pl.semaphore_signal(
                barrier_sem, inc=1,
                device_id=(nbr,), device_id_type=pl.DeviceIdType.MESH,
            )
        pl.semaphore_wait(barrier_sem, 2)

        def partial(c, col0):
            return jnp.dot(
                a_ref[pl.ds(c * mc, mc), :], b_ref[:, pl.ds(col0, nh)],
                preferred_element_type=jnp.float32,
            )

        def rdma_pair(s, cw_buf, ccw_buf, cws, cwr, ccws, ccwr):
            ss = s % 2
            rr = (s + 1) % 2
            cw = pltpu.make_async_remote_copy(
                src_ref=cw_buf.at[ss], dst_ref=cw_buf.at[rr],
                send_sem=cws.at[ss], recv_sem=cwr.at[rr],
                device_id=(right,), device_id_type=pl.DeviceIdType.MESH,
            )
            ccw = pltpu.make_async_remote_copy(
                src_ref=ccw_buf.at[ss], dst_ref=ccw_buf.at[rr],
                send_sem=ccws.at[ss], recv_sem=ccwr.at[rr],
                device_id=(left,), device_id_type=pl.DeviceIdType.MESH,
            )
            return cw, ccw, rr

        cw_comm[0] = partial(my, 0)
        ccw_comm[0] = partial(my, nh)
        for s in range(N_DEV - 1):
            cw, ccw, rr = rdma_pair(
                s, cw_comm, ccw_comm, cw_send, cw_recv, ccw_send, ccw_recv)
            cw.start()
            ccw.start()
            p_cw = partial((my - s - 1) % N_DEV, 0)
            p_ccw = partial((my + s + 1) % N_DEV, nh)
            cw.wait()
            cw_comm[rr] += p_cw
            ccw.wait()
            ccw_comm[rr] += p_ccw

        last = (N_DEV - 1) % 2
        own_cw = (my + 1) % N_DEV
        own_ccw = (my + N_DEV - 1) % N_DEV
        g_cw = _gelu(cw_comm[last])
        out_ref[pl.ds(own_cw * mc, mc), pl.ds(0, nh)] = g_cw
        agcw_comm[0] = g_cw
        g_ccw = _gelu(ccw_comm[last])
        out_ref[pl.ds(own_ccw * mc, mc), pl.ds(nh, nh)] = g_ccw
        agccw_comm[0] = g_ccw

        for t in range(N_DEV - 1):
            cw, ccw, rr = rdma_pair(
                t, agcw_comm, agccw_comm,
                agcw_send, agcw_recv, agccw_send, agccw_recv)
            cw.start()
            ccw.start()
            cw.wait()
            out_ref[pl.ds(((my - t) % N_DEV) * mc, mc), pl.ds(0, nh)] = (
                agcw_comm[rr])
            ccw.wait()
            out_ref[pl.ds(((my + t) % N_DEV) * mc, mc), pl.ds(nh, nh)] = (
                agccw_comm[rr])

    return pl.pallas_call(
        body,
        out_shape=jax.ShapeDtypeStruct((m, n), jnp.float32),
        in_specs=[
            pl.BlockSpec(memory_space=pltpu.VMEM),
            pl.BlockSpec(memory_space=pltpu.VMEM),
        ],
        out_specs=pl.BlockSpec(memory_space=pltpu.VMEM),
        scratch_shapes=[
            pltpu.VMEM((2, mc, nh), jnp.float32),
            pltpu.VMEM((2, mc, nh), jnp.float32),
            pltpu.VMEM((2, mc, nh), jnp.float32),
            pltpu.VMEM((2, mc, nh), jnp.float32),
            pltpu.SemaphoreType.DMA((2,)),
            pltpu.SemaphoreType.DMA((2,)),
            pltpu.SemaphoreType.DMA((2,)),
            pltpu.SemaphoreType.DMA((2,)),
            pltpu.SemaphoreType.DMA((2,)),
            pltpu.SemaphoreType.DMA((2,)),
            pltpu.SemaphoreType.DMA((2,)),
            pltpu.SemaphoreType.DMA((2,)),
        ],
        compiler_params=pltpu.CompilerParams(
            collective_id=0,
            vmem_limit_bytes=100 * 1024 * 1024,
        ),
    )(A, B)


# device time: 111038 ns/iter; 2.8160x vs baseline; 1.6021x over previous
import jax
import jax.numpy as jnp
from jax import lax
from jax.experimental import pallas as pl
from jax.experimental.pallas import tpu as pltpu

N_DEV = 4


def _gelu(z):
    return 0.5 * z * (1.0 + jnp.tanh(0.7978845608 * (z + 0.044715 * z * z * z)))


def kernel(A, B):
    m, k = A.shape
    _, n = B.shape
    mc = m // N_DEV
    nh = n // 2

    def body(a_ref, b_ref, out_ref, cw_comm, ccw_comm, agcw_comm, agccw_comm,
             cw_send, cw_recv, ccw_send, ccw_recv,
             agcw_send, agcw_recv, agccw_send, agccw_recv):
        my = lax.axis_index("i")
        left = (my + N_DEV - 1) % N_DEV
        right = (my + 1) % N_DEV

        barrier_sem = pltpu.get_barrier_semaphore()
        for nbr in (left, right):
            pl.semaphore_signal(
                barrier_sem, inc=1,
                device_id=(nbr,), device_id_type=pl.DeviceIdType.MESH,
            )
        pl.semaphore_wait(barrier_sem, 2)

        def partial(c, col0):
            return jnp.dot(
                a_ref[pl.ds(c * mc, mc), :], b_ref[:, pl.ds(col0, nh)],
                preferred_element_type=jnp.float32,
            )

        def rdma_pair(s, cw_buf, ccw_buf, cws, cwr, ccws, ccwr):
            ss = s % 2
            rr = (s + 1) % 2
            cw = pltpu.make_async_remote_copy(
                src_ref=cw_buf.at[ss], dst_ref=cw_buf.at[rr],
                send_sem=cws.at[ss], recv_sem=cwr.at[rr],
                device_id=(right,), device_id_type=pl.DeviceIdType.MESH,
            )
            ccw = pltpu.make_async_remote_copy(
                src_ref=ccw_buf.at[ss], dst_ref=ccw_buf.at[rr],
                send_sem=ccws.at[ss], recv_sem=ccwr.at[rr],
                device_id=(left,), device_id_type=pl.DeviceIdType.MESH,
            )
            return cw, ccw, rr

        cw_comm[0] = partial(my, 0).astype(jnp.bfloat16)
        ccw_comm[0] = partial(my, nh).astype(jnp.bfloat16)
        for s in range(N_DEV - 1):
            cw, ccw, rr = rdma_pair(
                s, cw_comm, ccw_comm, cw_send, cw_recv, ccw_send, ccw_recv)
            cw.start()
            ccw.start()
            p_cw = partial((my - s - 1) % N_DEV, 0)
            p_ccw = partial((my + s + 1) % N_DEV, nh)
            cw.wait()
            cw_comm[rr] = (
                cw_comm[rr].astype(jnp.float32) + p_cw
            ).astype(jnp.bfloat16)
            ccw.wait()
            ccw_comm[rr] = (
                ccw_comm[rr].astype(jnp.float32) + p_ccw
            ).astype(jnp.bfloat16)

        last = (N_DEV - 1) % 2
        own_cw = (my + 1) % N_DEV
        own_ccw = (my + N_DEV - 1) % N_DEV
        g_cw = _gelu(cw_comm[last].astype(jnp.float32))
        out_ref[pl.ds(own_cw * mc, mc), pl.ds(0, nh)] = g_cw
        agcw_comm[0] = g_cw.astype(jnp.bfloat16)
        g_ccw = _gelu(ccw_comm[last].astype(jnp.float32))
        out_ref[pl.ds(own_ccw * mc, mc), pl.ds(nh, nh)] = g_ccw
        agccw_comm[0] = g_ccw.astype(jnp.bfloat16)

        for t in range(N_DEV - 1):
            cw, ccw, rr = rdma_pair(
                t, agcw_comm, agccw_comm,
                agcw_send, agcw_recv, agccw_send, agccw_recv)
            cw.start()
            ccw.start()
            cw.wait()
            out_ref[pl.ds(((my - t) % N_DEV) * mc, mc), pl.ds(0, nh)] = (
                agcw_comm[rr].astype(jnp.float32))
            ccw.wait()
            out_ref[pl.ds(((my + t) % N_DEV) * mc, mc), pl.ds(nh, nh)] = (
                agccw_comm[rr].astype(jnp.float32))

    return pl.pallas_call(
        body,
        out_shape=jax.ShapeDtypeStruct((m, n), jnp.float32),
        in_specs=[
            pl.BlockSpec(memory_space=pltpu.VMEM),
            pl.BlockSpec(memory_space=pltpu.VMEM),
        ],
        out_specs=pl.BlockSpec(memory_space=pltpu.VMEM),
        scratch_shapes=[
            pltpu.VMEM((2, mc, nh), jnp.bfloat16),
            pltpu.VMEM((2, mc, nh), jnp.bfloat16),
            pltpu.VMEM((2, mc, nh), jnp.bfloat16),
            pltpu.VMEM((2, mc, nh), jnp.bfloat16),
            pltpu.SemaphoreType.DMA((2,)),
            pltpu.SemaphoreType.DMA((2,)),
            pltpu.SemaphoreType.DMA((2,)),
            pltpu.SemaphoreType.DMA((2,)),
            pltpu.SemaphoreType.DMA((2,)),
            pltpu.SemaphoreType.DMA((2,)),
            pltpu.SemaphoreType.DMA((2,)),
            pltpu.SemaphoreType.DMA((2,)),
        ],
        compiler_params=pltpu.CompilerParams(
            collective_id=0,
            vmem_limit_bytes=100 * 1024 * 1024,
        ),
    )(A, B)


# device time: 27141 ns/iter; 11.5208x vs baseline; 4.0912x over previous
import jax
import jax.numpy as jnp
from jax import lax
from jax.experimental import pallas as pl
from jax.experimental.pallas import tpu as pltpu

N_DEV = 4


def _gelu(z):
    return 0.5 * z * (1.0 + jnp.tanh(0.7978845608 * (z + 0.044715 * z * z * z)))


def kernel(A, B):
    m, k = A.shape
    _, n = B.shape
    mc = m // N_DEV
    nh = n // 2

    def body(a_ref, b_ref, out_ref, a_bf, b_bf,
             cw_comm, ccw_comm, agcw_comm, agccw_comm,
             cw_send, cw_recv, ccw_send, ccw_recv,
             agcw_send, agcw_recv, agccw_send, agccw_recv):
        my = lax.axis_index("i")
        left = (my + N_DEV - 1) % N_DEV
        right = (my + 1) % N_DEV

        a_bf[...] = a_ref[...].astype(jnp.bfloat16)
        b_bf[...] = b_ref[...].astype(jnp.bfloat16)

        barrier_sem = pltpu.get_barrier_semaphore()
        for nbr in (left, right):
            pl.semaphore_signal(
                barrier_sem, inc=1,
                device_id=(nbr,), device_id_type=pl.DeviceIdType.MESH,
            )
        pl.semaphore_wait(barrier_sem, 2)

        def partial(c, col0):
            return jnp.dot(
                a_bf[pl.ds(c * mc, mc), :], b_bf[:, pl.ds(col0, nh)],
                preferred_element_type=jnp.float32,
            )

        def rdma_pair(s, cw_buf, ccw_buf, cws, cwr, ccws, ccwr):
            ss = s % 2
            rr = (s + 1) % 2
            cw = pltpu.make_async_remote_copy(
                src_ref=cw_buf.at[ss], dst_ref=cw_buf.at[rr],
                send_sem=cws.at[ss], recv_sem=cwr.at[rr],
                device_id=(right,), device_id_type=pl.DeviceIdType.MESH,
            )
            ccw = pltpu.make_async_remote_copy(
                src_ref=ccw_buf.at[ss], dst_ref=ccw_buf.at[rr],
                send_sem=ccws.at[ss], recv_sem=ccwr.at[rr],
                device_id=(left,), device_id_type=pl.DeviceIdType.MESH,
            )
            return cw, ccw, rr

        cw_comm[0] = partial(my, 0).astype(jnp.bfloat16)
        ccw_comm[0] = partial(my, nh).astype(jnp.bfloat16)
        for s in range(N_DEV - 1):
            cw, ccw, rr = rdma_pair(
                s, cw_comm, ccw_comm, cw_send, cw_recv, ccw_send, ccw_recv)
            cw.start()
            ccw.start()
            p_cw = partial((my - s - 1) % N_DEV, 0)
            p_ccw = partial((my + s + 1) % N_DEV, nh)
            cw.wait()
            cw_comm[rr] = (
                cw_comm[rr].astype(jnp.float32) + p_cw
            ).astype(jnp.bfloat16)
            ccw.wait()
            ccw_comm[rr] = (
                ccw_comm[rr].astype(jnp.float32) + p_ccw
            ).astype(jnp.bfloat16)

        last = (N_DEV - 1) % 2
        own_cw = (my + 1) % N_DEV
        own_ccw = (my + N_DEV - 1) % N_DEV
        g_cw = _gelu(cw_comm[last].astype(jnp.float32))
        out_ref[pl.ds(own_cw * mc, mc), pl.ds(0, nh)] = g_cw
        agcw_comm[0] = g_cw.astype(jnp.bfloat16)
        g_ccw = _gelu(ccw_comm[last].astype(jnp.float32))
        out_ref[pl.ds(own_ccw * mc, mc), pl.ds(nh, nh)] = g_ccw
        agccw_comm[0] = g_ccw.astype(jnp.bfloat16)

        for t in range(N_DEV - 1):
            cw, ccw, rr = rdma_pair(
                t, agcw_comm, agccw_comm,
                agcw_send, agcw_recv, agccw_send, agccw_recv)
            cw.start()
            ccw.start()
            cw.wait()
            out_ref[pl.ds(((my - t) % N_DEV) * mc, mc), pl.ds(0, nh)] = (
                agcw_comm[rr].astype(jnp.float32))
            ccw.wait()
            out_ref[pl.ds(((my + t) % N_DEV) * mc, mc), pl.ds(nh, nh)] = (
                agccw_comm[rr].astype(jnp.float32))

    return pl.pallas_call(
        body,
        out_shape=jax.ShapeDtypeStruct((m, n), jnp.float32),
        in_specs=[
            pl.BlockSpec(memory_space=pltpu.VMEM),
            pl.BlockSpec(memory_space=pltpu.VMEM),
        ],
        out_specs=pl.BlockSpec(memory_space=pltpu.VMEM),
        scratch_shapes=[
            pltpu.VMEM((m, k), jnp.bfloat16),
            pltpu.VMEM((k, n), jnp.bfloat16),
            pltpu.VMEM((2, mc, nh), jnp.bfloat16),
            pltpu.VMEM((2, mc, nh), jnp.bfloat16),
            pltpu.VMEM((2, mc, nh), jnp.bfloat16),
            pltpu.VMEM((2, mc, nh), jnp.bfloat16),
            pltpu.SemaphoreType.DMA((2,)),
            pltpu.SemaphoreType.DMA((2,)),
            pltpu.SemaphoreType.DMA((2,)),
            pltpu.SemaphoreType.DMA((2,)),
            pltpu.SemaphoreType.DMA((2,)),
            pltpu.SemaphoreType.DMA((2,)),
            pltpu.SemaphoreType.DMA((2,)),
            pltpu.SemaphoreType.DMA((2,)),
        ],
        compiler_params=pltpu.CompilerParams(
            collective_id=0,
            vmem_limit_bytes=100 * 1024 * 1024,
        ),
    )(A, B)
